# baseline (speedup 1.0000x reference)
import jax
import jax.numpy as jnp
from jax.experimental import pallas as pl
from jax.experimental.pallas import tpu as pltpu

_SEQ = 8192
_DIM = 1024
_NSLICE = 8
_ROWS = _SEQ // _NSLICE


def _dma_body(in_hbm, out_hbm, *sems):
    copies = []
    for i in range(_NSLICE):
        c = pltpu.make_async_copy(
            in_hbm.at[pl.ds(i * _ROWS, _ROWS)],
            out_hbm.at[pl.ds(i * _ROWS, _ROWS)],
            sems[i],
        )
        c.start()
        copies.append(c)
    for c in copies:
        c.wait()


def kernel(hidden_embs, position_embedding_table):
    del hidden_embs
    return pl.pallas_call(
        _dma_body,
        in_specs=[pl.BlockSpec(memory_space=pl.ANY)],
        out_specs=pl.BlockSpec(memory_space=pl.ANY),
        scratch_shapes=[pltpu.SemaphoreType.DMA] * _NSLICE,
        out_shape=jax.ShapeDtypeStruct((_SEQ, _DIM), jnp.float32),
    )(position_embedding_table)


# speedup vs baseline: 22.3033x; 22.3033x over previous
import functools

import jax
import jax.numpy as jnp
from jax import lax
from jax.experimental import pallas as pl
from jax.experimental.pallas import tpu as pltpu
from jax.experimental.pallas import tpu_sc as plsc

_SEQ = 8192
_DIM = 1024

_info = plsc.get_sparse_core_info()
_NC, _NS = _info.num_cores, _info.num_subcores
_NW = _NC * _NS
_ROWS_PER_W = _SEQ // _NW

_CH = 16
_NCHUNK = _ROWS_PER_W // _CH
_NB = 7

_mesh = plsc.VectorSubcoreMesh(core_axis_name="c", subcore_axis_name="s")


@functools.partial(
    pl.kernel,
    mesh=_mesh,
    out_type=jax.ShapeDtypeStruct((_SEQ, _DIM), jnp.float32),
    scratch_types=(
        [pltpu.VMEM((_CH, _DIM), jnp.float32) for _ in range(_NB)]
        + [pltpu.SemaphoreType.DMA for _ in range(_NB)]
        + [pltpu.SemaphoreType.DMA for _ in range(_NB)]
    ),
)
def _pos_emb_copy(table_hbm, out_hbm, *scratch):
    bufs = scratch[:_NB]
    rsems = scratch[_NB:2 * _NB]
    wsems = scratch[2 * _NB:]

    wid = lax.axis_index("s") * _NC + lax.axis_index("c")
    base = wid * _ROWS_PER_W

    reads = [None] * _NCHUNK
    writes = [None] * _NCHUNK

    for i in range(min(_NB, _NCHUNK)):
        reads[i] = pltpu.async_copy(
            table_hbm.at[pl.ds(base + i * _CH, _CH)], bufs[i], rsems[i]
        )
    for i in range(_NCHUNK):
        reads[i].wait()
        writes[i] = pltpu.async_copy(
            bufs[i % _NB], out_hbm.at[pl.ds(base + i * _CH, _CH)], wsems[i % _NB]
        )
        j = i + 1
        if _NB <= j < _NCHUNK:
            writes[j - _NB].wait()
            reads[j] = pltpu.async_copy(
                table_hbm.at[pl.ds(base + j * _CH, _CH)], bufs[j % _NB], rsems[j % _NB]
            )
    for i in range(max(0, _NCHUNK - _NB), _NCHUNK):
        writes[i].wait()


def kernel(hidden_embs, position_embedding_table):
    del hidden_embs
    return _pos_emb_copy(position_embedding_table)


# speedup vs baseline: 23.5837x; 1.0574x over previous
import functools

import jax
import jax.numpy as jnp
from jax import lax
from jax.experimental import pallas as pl
from jax.experimental.pallas import tpu as pltpu
from jax.experimental.pallas import tpu_sc as plsc

_SEQ = 8192
_DIM = 1024

_info = plsc.get_sparse_core_info()
_NC, _NS = _info.num_cores, _info.num_subcores
_NW = _NC * _NS
_ROWS_PER_W = _SEQ // _NW

_CH = 32
_NCHUNK = _ROWS_PER_W // _CH
_NB = 4

_mesh = plsc.VectorSubcoreMesh(core_axis_name="c", subcore_axis_name="s")


@functools.partial(
    pl.kernel,
    mesh=_mesh,
    out_type=jax.ShapeDtypeStruct((_SEQ, _DIM), jnp.float32),
    scratch_types=(
        [pltpu.VMEM((_CH, _DIM), jnp.float32) for _ in range(_NB)]
        + [pltpu.SemaphoreType.DMA for _ in range(_NB)]
        + [pltpu.SemaphoreType.DMA for _ in range(_NB)]
    ),
)
def _pos_emb_copy(table_hbm, out_hbm, *scratch):
    bufs = scratch[:_NB]
    rsems = scratch[_NB:2 * _NB]
    wsems = scratch[2 * _NB:]

    wid = lax.axis_index("s") * _NC + lax.axis_index("c")
    base = wid * _ROWS_PER_W

    reads = [None] * _NCHUNK
    writes = [None] * _NCHUNK

    for i in range(min(_NB, _NCHUNK)):
        reads[i] = pltpu.async_copy(
            table_hbm.at[pl.ds(base + i * _CH, _CH)], bufs[i], rsems[i]
        )
    for i in range(_NCHUNK):
        reads[i].wait()
        writes[i] = pltpu.async_copy(
            bufs[i % _NB], out_hbm.at[pl.ds(base + i * _CH, _CH)], wsems[i % _NB]
        )
        j = i + 1
        if _NB <= j < _NCHUNK:
            writes[j - _NB].wait()
            reads[j] = pltpu.async_copy(
                table_hbm.at[pl.ds(base + j * _CH, _CH)], bufs[j % _NB], rsems[j % _NB]
            )
    for i in range(max(0, _NCHUNK - _NB), _NCHUNK):
        writes[i].wait()


def kernel(hidden_embs, position_embedding_table):
    del hidden_embs
    return _pos_emb_copy(position_embedding_table)


# speedup vs baseline: 24.0241x; 1.0187x over previous
import functools

import jax
import jax.numpy as jnp
from jax import lax
from jax.experimental import pallas as pl
from jax.experimental.pallas import tpu as pltpu
from jax.experimental.pallas import tpu_sc as plsc

_SEQ = 8192
_DIM = 1024

_info = plsc.get_sparse_core_info()
_NC, _NS = _info.num_cores, _info.num_subcores
_NW = _NC * _NS
_ROWS_PER_W = _SEQ // _NW

_CH = 32
_NCHUNK = _ROWS_PER_W // _CH
_NB = 3

_mesh = plsc.VectorSubcoreMesh(core_axis_name="c", subcore_axis_name="s")


@functools.partial(
    pl.kernel,
    mesh=_mesh,
    out_type=jax.ShapeDtypeStruct((_SEQ, _DIM), jnp.float32),
    scratch_types=(
        [pltpu.VMEM_SHARED((_NS, _CH, _DIM), jnp.float32)
         for _ in range(_NB)]
        + [pltpu.SemaphoreType.DMA for _ in range(_NB)]
        + [pltpu.SemaphoreType.DMA for _ in range(_NB)]
    ),
)
def _pos_emb_copy(table_hbm, out_hbm, *scratch):
    bufs = scratch[:_NB]
    rsems = scratch[_NB:2 * _NB]
    wsems = scratch[2 * _NB:]

    sid = lax.axis_index("s")
    wid = sid * _NC + lax.axis_index("c")
    base = wid * _ROWS_PER_W

    reads = [None] * _NCHUNK
    writes = [None] * _NCHUNK

    for i in range(min(_NB, _NCHUNK)):
        reads[i] = pltpu.async_copy(
            table_hbm.at[pl.ds(base + i * _CH, _CH)], bufs[i].at[sid], rsems[i]
        )
    for i in range(_NCHUNK):
        b = i % _NB
        reads[i].wait()
        writes[i] = pltpu.async_copy(
            bufs[b].at[sid], out_hbm.at[pl.ds(base + i * _CH, _CH)], wsems[b]
        )
        j = i + 1
        if _NB <= j < _NCHUNK:
            writes[j - _NB].wait()
            reads[j] = pltpu.async_copy(
                table_hbm.at[pl.ds(base + j * _CH, _CH)], bufs[j % _NB].at[sid],
                rsems[j % _NB]
            )
    for i in range(max(0, _NCHUNK - _NB), _NCHUNK):
        writes[i].wait()


def kernel(hidden_embs, position_embedding_table):
    del hidden_embs
    return _pos_emb_copy(position_embedding_table)
